# baseline (speedup 1.0000x reference)
import functools

import jax
import jax.numpy as jnp
from jax import lax
from jax.experimental import pallas as pl
from jax.experimental.pallas import tpu as pltpu
from jax.experimental.pallas import tpu_sc as plsc

_PC_START = (-54.0, -54.0)
_VOXEL_SIZE = (0.075, 0.075)
_OUT_STRIDE = 8

_LANES = 16
_RL = 128


@functools.cache
def _make_sc_kernel(B, C, H, W, N, P):
    info = plsc.get_sparse_core_info()
    NC, NS = info.num_cores, info.num_subcores
    NW = NC * NS
    NPTS = N * P
    CT = C // _RL
    TN = N // 8
    TPC = P * C // _RL
    ppw = B * NPTS // NW
    nchunk = ppw // _LANES
    wpb = NW // B
    npw = N // wpb
    assert CT * _RL == C and TN * 8 == N and wpb * B == NW
    assert ppw * NW == B * NPTS and nchunk % 2 == 0
    assert npw == _RL and nchunk == P * (npw // _LANES)

    sx = float(_VOXEL_SIZE[0] * _OUT_STRIDE)
    sy = float(_VOXEL_SIZE[1] * _OUT_STRIDE)
    ox = float(_PC_START[0])
    oy = float(_PC_START[1])

    mesh = plsc.VectorSubcoreMesh(core_axis_name="c", subcore_axis_name="s")

    @functools.partial(
        pl.kernel,
        out_type=jax.ShapeDtypeStruct((B * TN * TPC * 8, _RL), jnp.float32),
        mesh=mesh,
        compiler_params=pltpu.CompilerParams(use_tc_tiling_on_sc=False,
                                             needs_layout_passes=False),
        scratch_types=[
            pltpu.VMEM((P, 2, npw), jnp.float32),
            pltpu.VMEM((ppw,), jnp.float32),
            pltpu.VMEM((ppw,), jnp.float32),
            pltpu.VMEM((ppw,), jnp.float32),
            pltpu.VMEM((ppw,), jnp.float32),
            pltpu.VMEM((nchunk // 2, 16 * _LANES), jnp.int32),
            pltpu.VMEM((nchunk // 2, 4 * _LANES), jnp.int32),
            pltpu.VMEM((16 * _LANES, _RL), jnp.float32),
            pltpu.VMEM((16 * _LANES, _RL), jnp.float32),
            pltpu.VMEM((4 * _LANES, _RL), jnp.float32),
            pltpu.VMEM((4 * _LANES, _RL), jnp.float32),
            pltpu.SemaphoreType.DMA,
            pltpu.SemaphoreType.DMA,
            pltpu.SemaphoreType.DMA,
            pltpu.SemaphoreType.DMA,
        ],
    )
    def bev_kernel(rows_hbm, bc_hbm, out_hbm,
                   bcb, wab, wbb, wcb, wdb, gidx, oidx,
                   gb0, gb1, ob0, ob1, gs0, gs1, os0, os1):
        wid = lax.axis_index("s") * NC + lax.axis_index("c")
        b = wid // wpb
        nb = wid % wpb
        n0 = nb * npw

        pltpu.sync_copy(bc_hbm.at[b, :, nb], bcb)

        lane = lax.iota(jnp.int32, _LANES)

        nc_per_p = npw // _LANES

        def stage_a(ci, _):
            p = ci // nc_per_p
            nc = ci - p * nc_per_p
            sn = pl.ds(nc * _LANES, _LANES)
            x = (bcb[p, 0, sn] - ox) / sx
            y = (bcb[p, 1, sn] - oy) / sy
            xi = x.astype(jnp.int32)
            yi = y.astype(jnp.int32)
            xi = jnp.where(xi.astype(jnp.float32) > x, xi - 1, xi)
            yi = jnp.where(yi.astype(jnp.float32) > y, yi - 1, yi)
            x0 = jnp.clip(xi, 0, W - 1)
            y0 = jnp.clip(yi, 0, H - 1)
            x1 = jnp.minimum(x0 + 1, W - 1)
            y1 = jnp.minimum(y0 + 1, H - 1)
            wx0 = x - x0.astype(jnp.float32)
            wx1 = x1.astype(jnp.float32) - x
            wy0 = y - y0.astype(jnp.float32)
            wy1 = y1.astype(jnp.float32) - y
            s = pl.ds(ci * _LANES, _LANES)
            wab[s] = wx1 * wy1
            wbb[s] = wx1 * wy0
            wcb[s] = wx0 * wy1
            wdb[s] = wx0 * wy0
            ra = (y0 * W + x0) * (CT * B) + b
            rb = (y1 * W + x0) * (CT * B) + b
            rc = (y0 * W + x1) * (CT * B) + b
            rd = (y1 * W + x1) * (CT * B) + b
            c2 = ci // 2
            g0 = (ci - c2 * 2) * (8 * _LANES)
            gidx[c2, pl.ds(g0 + 0 * _LANES, _LANES)] = ra
            gidx[c2, pl.ds(g0 + 1 * _LANES, _LANES)] = ra + B
            gidx[c2, pl.ds(g0 + 2 * _LANES, _LANES)] = rb
            gidx[c2, pl.ds(g0 + 3 * _LANES, _LANES)] = rb + B
            gidx[c2, pl.ds(g0 + 4 * _LANES, _LANES)] = rc
            gidx[c2, pl.ds(g0 + 5 * _LANES, _LANES)] = rc + B
            gidx[c2, pl.ds(g0 + 6 * _LANES, _LANES)] = rd
            gidx[c2, pl.ds(g0 + 7 * _LANES, _LANES)] = rd + B
            n = n0 + nc * _LANES + lane
            o0 = ((b * TN + lax.shift_right_logical(n, 3)) * TPC
                  + p * CT) * 8 + (n & 7)
            q0 = (ci - c2 * 2) * (2 * _LANES)
            oidx[c2, pl.ds(q0, _LANES)] = o0
            oidx[c2, pl.ds(q0 + _LANES, _LANES)] = o0 + 8

        def combine(c2, gbuf, obuf):
            for sub in range(2):
                s = pl.ds((c2 * 2 + sub) * _LANES, _LANES)
                wa = wab[s]
                wb = wbb[s]
                wc = wcb[s]
                wd = wdb[s]
                gbase = sub * (8 * _LANES)
                obase = sub * (2 * _LANES)

                @plsc.parallel_loop(0, _LANES, step=1, unroll=2)
                def point(i, wa=wa, wb=wb, wc=wc, wd=wd,
                          gbase=gbase, obase=obase):
                    bi = jnp.full((_LANES,), i, jnp.int32)
                    wai = wa.at[bi].get(mode="promise_in_bounds")
                    wbi = wb.at[bi].get(mode="promise_in_bounds")
                    wci = wc.at[bi].get(mode="promise_in_bounds")
                    wdi = wd.at[bi].get(mode="promise_in_bounds")
                    for t in range(CT):
                        r = t * _LANES + i
                        for v in range(_RL // _LANES):
                            cs = pl.ds(v * _LANES, _LANES)
                            acc = (gbuf[gbase + 0 * CT * _LANES + r, cs] * wai
                                   + gbuf[gbase + 1 * CT * _LANES + r, cs] * wbi
                                   + gbuf[gbase + 2 * CT * _LANES + r, cs] * wci
                                   + gbuf[gbase + 3 * CT * _LANES + r, cs] * wdi)
                            obuf[obase + r, cs] = acc

        nbig = nchunk // 2
        last = nbig - 1
        for si in range(4):
            stage_a(si, None)
        pltpu.async_copy(rows_hbm.at[gidx.at[0]], gb0, gs0)
        pltpu.async_copy(rows_hbm.at[gidx.at[1]], gb1, gs1)

        def pair(hi, _):
            for par, gbuf, gsem, obuf, osem in (
                    (0, gb0, gs0, ob0, os0), (1, gb1, gs1, ob1, os1)):
                c2 = hi * 2 + par
                pltpu.make_async_copy(rows_hbm.at[gidx.at[c2]], gbuf,
                                      gsem).wait()

                @pl.when(hi > 0)
                def _wait_prev_scatter(obuf=obuf, osem=osem, c2=c2):
                    pltpu.make_async_copy(obuf, out_hbm.at[oidx.at[c2 - 2]],
                                          osem).wait()

                combine(c2, gbuf, obuf)
                pltpu.async_copy(obuf, out_hbm.at[oidx.at[c2]], osem)

                @pl.when(c2 + 2 <= last)
                def _prep_next(c2=c2):
                    stage_a((c2 + 2) * 2, None)
                    stage_a((c2 + 2) * 2 + 1, None)

                nxt = jnp.minimum(c2 + 2, last)
                pltpu.async_copy(rows_hbm.at[gidx.at[nxt]], gbuf, gsem)
            return 0

        lax.fori_loop(0, nbig // 2, pair, 0)

        pltpu.make_async_copy(rows_hbm.at[gidx.at[last]], gb0, gs0).wait()
        pltpu.make_async_copy(rows_hbm.at[gidx.at[last]], gb1, gs1).wait()
        pltpu.make_async_copy(ob0, out_hbm.at[oidx.at[last - 1]], os0).wait()
        pltpu.make_async_copy(ob1, out_hbm.at[oidx.at[last]], os1).wait()

    return bev_kernel


def kernel(bev_feature, batch_centers, num_point):
    if isinstance(num_point, tuple):
        num_point = num_point[0] * num_point[1]
    B, C, H, W = bev_feature.shape
    _, N, P, _ = batch_centers.shape
    CT = C // _RL
    TN = N // 8
    TPC = P * C // _RL
    bev_rows = jnp.transpose(
        bev_feature.reshape(B, CT, _RL, H, W), (3, 4, 1, 0, 2)
    ).reshape(H * W * CT * B, _RL)
    bc_view = jnp.transpose(
        batch_centers.reshape(B, N // _RL, _RL, P, 2), (0, 3, 1, 4, 2))
    out_rows = _make_sc_kernel(B, C, H, W, N, P)(bev_rows, bc_view)
    return jnp.transpose(
        out_rows.reshape(B, TN, TPC, 8, _RL), (0, 1, 3, 2, 4)
    ).reshape(B, N, P * C)


# speedup vs baseline: 1.2172x; 1.2172x over previous
import functools

import jax
import jax.numpy as jnp
from jax import lax
from jax.experimental import pallas as pl
from jax.experimental.pallas import tpu as pltpu
from jax.experimental.pallas import tpu_sc as plsc

_PC_START = (-54.0, -54.0)
_VOXEL_SIZE = (0.075, 0.075)
_OUT_STRIDE = 8

_LANES = 16
_RL = 128


@functools.cache
def _make_sc_kernel(B, C, H, W, N, P):
    info = plsc.get_sparse_core_info()
    NC, NS = info.num_cores, info.num_subcores
    NW = NC * NS
    NPTS = N * P
    CT = C // _RL
    TN = N // 8
    TPC = P * C // _RL
    ppw = B * NPTS // NW
    nchunk = ppw // _LANES
    wpb = NW // B
    npw = N // wpb
    assert CT * _RL == C and TN * 8 == N and wpb * B == NW
    assert ppw * NW == B * NPTS and nchunk % 2 == 0
    assert npw == _RL and nchunk == P * (npw // _LANES)

    sx = float(_VOXEL_SIZE[0] * _OUT_STRIDE)
    sy = float(_VOXEL_SIZE[1] * _OUT_STRIDE)
    ox = float(_PC_START[0])
    oy = float(_PC_START[1])

    mesh = plsc.VectorSubcoreMesh(core_axis_name="c", subcore_axis_name="s")

    @functools.partial(
        pl.kernel,
        out_type=jax.ShapeDtypeStruct((B * TN * TPC * 8, _RL), jnp.float32),
        mesh=mesh,
        compiler_params=pltpu.CompilerParams(use_tc_tiling_on_sc=False,
                                             needs_layout_passes=False),
        scratch_types=[
            pltpu.VMEM((P, 2, npw), jnp.float32),
            pltpu.VMEM((ppw,), jnp.float32),
            pltpu.VMEM((ppw,), jnp.float32),
            pltpu.VMEM((ppw,), jnp.float32),
            pltpu.VMEM((ppw,), jnp.float32),
            pltpu.VMEM((nchunk // 2, 16 * _LANES), jnp.int32),
            pltpu.VMEM((nchunk // 2, 4 * _LANES), jnp.int32),
            pltpu.VMEM((16 * _LANES, _RL), jnp.float32),
            pltpu.VMEM((16 * _LANES, _RL), jnp.float32),
            pltpu.VMEM((4 * _LANES, _RL), jnp.float32),
            pltpu.VMEM((4 * _LANES, _RL), jnp.float32),
            pltpu.SemaphoreType.DMA,
            pltpu.SemaphoreType.DMA,
            pltpu.SemaphoreType.DMA,
            pltpu.SemaphoreType.DMA,
        ],
    )
    def bev_kernel(rows_hbm, bc_hbm, out_hbm,
                   bcb, wab, wbb, wcb, wdb, gidx, oidx,
                   gb0, gb1, ob0, ob1, gs0, gs1, os0, os1):
        wid = lax.axis_index("s") * NC + lax.axis_index("c")
        b = wid // wpb
        nb = wid % wpb
        n0 = nb * npw

        pltpu.sync_copy(bc_hbm.at[b, :, nb], bcb)

        lane = lax.iota(jnp.int32, _LANES)

        nc_per_p = npw // _LANES

        def stage_a(ci, _):
            p = ci // nc_per_p
            nc = ci - p * nc_per_p
            sn = pl.ds(nc * _LANES, _LANES)
            x = (bcb[p, 0, sn] - ox) / sx
            y = (bcb[p, 1, sn] - oy) / sy
            xi = x.astype(jnp.int32)
            yi = y.astype(jnp.int32)
            xi = jnp.where(xi.astype(jnp.float32) > x, xi - 1, xi)
            yi = jnp.where(yi.astype(jnp.float32) > y, yi - 1, yi)
            x0 = jnp.clip(xi, 0, W - 1)
            y0 = jnp.clip(yi, 0, H - 1)
            x1 = jnp.minimum(x0 + 1, W - 1)
            y1 = jnp.minimum(y0 + 1, H - 1)
            wx0 = x - x0.astype(jnp.float32)
            wx1 = x1.astype(jnp.float32) - x
            wy0 = y - y0.astype(jnp.float32)
            wy1 = y1.astype(jnp.float32) - y
            s = pl.ds(ci * _LANES, _LANES)
            wab[s] = wx1 * wy1
            wbb[s] = wx1 * wy0
            wcb[s] = wx0 * wy1
            wdb[s] = wx0 * wy0
            ra = (y0 * W + x0) * (CT * B) + b
            rb = (y1 * W + x0) * (CT * B) + b
            rc = (y0 * W + x1) * (CT * B) + b
            rd = (y1 * W + x1) * (CT * B) + b
            c2 = ci // 2
            g0 = (ci - c2 * 2) * (8 * _LANES)
            gidx[c2, pl.ds(g0 + 0 * _LANES, _LANES)] = ra
            gidx[c2, pl.ds(g0 + 1 * _LANES, _LANES)] = ra + B
            gidx[c2, pl.ds(g0 + 2 * _LANES, _LANES)] = rb
            gidx[c2, pl.ds(g0 + 3 * _LANES, _LANES)] = rb + B
            gidx[c2, pl.ds(g0 + 4 * _LANES, _LANES)] = rc
            gidx[c2, pl.ds(g0 + 5 * _LANES, _LANES)] = rc + B
            gidx[c2, pl.ds(g0 + 6 * _LANES, _LANES)] = rd
            gidx[c2, pl.ds(g0 + 7 * _LANES, _LANES)] = rd + B
            n = n0 + nc * _LANES + lane
            o0 = ((b * TN + lax.shift_right_logical(n, 3)) * TPC
                  + p * CT) * 8 + (n & 7)
            q0 = (ci - c2 * 2) * (2 * _LANES)
            oidx[c2, pl.ds(q0, _LANES)] = o0
            oidx[c2, pl.ds(q0 + _LANES, _LANES)] = o0 + 8

        def combine(c2, gbuf, obuf):
            for sub in range(2):
                s = pl.ds((c2 * 2 + sub) * _LANES, _LANES)
                wa = wab[s]
                wb = wbb[s]
                wc = wcb[s]
                wd = wdb[s]
                gbase = sub * (8 * _LANES)
                obase = sub * (2 * _LANES)

                @plsc.parallel_loop(0, _LANES, step=1, unroll=1)
                def point(i, wa=wa, wb=wb, wc=wc, wd=wd,
                          gbase=gbase, obase=obase):
                    bi = jnp.full((_LANES,), i, jnp.int32)
                    wai = wa.at[bi].get(mode="promise_in_bounds")
                    wbi = wb.at[bi].get(mode="promise_in_bounds")
                    wci = wc.at[bi].get(mode="promise_in_bounds")
                    wdi = wd.at[bi].get(mode="promise_in_bounds")
                    for t in range(CT):
                        r = t * _LANES + i
                        for v in range(_RL // _LANES):
                            cs = pl.ds(v * _LANES, _LANES)
                            acc = (gbuf[gbase + 0 * CT * _LANES + r, cs] * wai
                                   + gbuf[gbase + 1 * CT * _LANES + r, cs] * wbi
                                   + gbuf[gbase + 2 * CT * _LANES + r, cs] * wci
                                   + gbuf[gbase + 3 * CT * _LANES + r, cs] * wdi)
                            obuf[obase + r, cs] = acc

        nbig = nchunk // 2
        last = nbig - 1
        for si in range(4):
            stage_a(si, None)
        pltpu.async_copy(rows_hbm.at[gidx.at[0]], gb0, gs0)
        pltpu.async_copy(rows_hbm.at[gidx.at[1]], gb1, gs1)

        def pair(hi, _):
            for par, gbuf, gsem, obuf, osem in (
                    (0, gb0, gs0, ob0, os0), (1, gb1, gs1, ob1, os1)):
                c2 = hi * 2 + par
                pltpu.make_async_copy(rows_hbm.at[gidx.at[c2]], gbuf,
                                      gsem).wait()

                @pl.when(hi > 0)
                def _wait_prev_scatter(obuf=obuf, osem=osem, c2=c2):
                    pltpu.make_async_copy(obuf, out_hbm.at[oidx.at[c2 - 2]],
                                          osem).wait()

                combine(c2, gbuf, obuf)
                pltpu.async_copy(obuf, out_hbm.at[oidx.at[c2]], osem)

                @pl.when(c2 + 2 <= last)
                def _prep_next(c2=c2):
                    stage_a((c2 + 2) * 2, None)
                    stage_a((c2 + 2) * 2 + 1, None)

                nxt = jnp.minimum(c2 + 2, last)
                pltpu.async_copy(rows_hbm.at[gidx.at[nxt]], gbuf, gsem)
            return 0

        lax.fori_loop(0, nbig // 2, pair, 0)

        pltpu.make_async_copy(rows_hbm.at[gidx.at[last]], gb0, gs0).wait()
        pltpu.make_async_copy(rows_hbm.at[gidx.at[last]], gb1, gs1).wait()
        pltpu.make_async_copy(ob0, out_hbm.at[oidx.at[last - 1]], os0).wait()
        pltpu.make_async_copy(ob1, out_hbm.at[oidx.at[last]], os1).wait()

    return bev_kernel


def kernel(bev_feature, batch_centers, num_point):
    if isinstance(num_point, tuple):
        num_point = num_point[0] * num_point[1]
    B, C, H, W = bev_feature.shape
    _, N, P, _ = batch_centers.shape
    CT = C // _RL
    TN = N // 8
    TPC = P * C // _RL
    bev_rows = jnp.transpose(
        bev_feature.reshape(B, CT, _RL, H, W), (3, 4, 1, 0, 2)
    ).reshape(H * W * CT * B, _RL)
    bc_view = jnp.transpose(
        batch_centers.reshape(B, N // _RL, _RL, P, 2), (0, 3, 1, 4, 2))
    out_rows = _make_sc_kernel(B, C, H, W, N, P)(bev_rows, bc_view)
    return jnp.transpose(
        out_rows.reshape(B, TN, TPC, 8, _RL), (0, 1, 3, 2, 4)
    ).reshape(B, N, P * C)


# speedup vs baseline: 1.2264x; 1.0075x over previous
import functools

import jax
import jax.numpy as jnp
from jax import lax
from jax.experimental import pallas as pl
from jax.experimental.pallas import tpu as pltpu
from jax.experimental.pallas import tpu_sc as plsc

_PC_START = (-54.0, -54.0)
_VOXEL_SIZE = (0.075, 0.075)
_OUT_STRIDE = 8

_LANES = 16
_RL = 128


@functools.cache
def _make_sc_kernel(B, C, H, W, N, P):
    info = plsc.get_sparse_core_info()
    NC, NS = info.num_cores, info.num_subcores
    NW = NC * NS
    NPTS = N * P
    CT = C // _RL
    TN = N // 8
    TPC = P * C // _RL
    ppw = B * NPTS // NW
    nchunk = ppw // _LANES
    wpb = NW // B
    npw = N // wpb
    assert CT * _RL == C and TN * 8 == N and wpb * B == NW
    assert ppw * NW == B * NPTS and nchunk % 2 == 0
    assert npw == _RL and nchunk == P * (npw // _LANES)

    sx = float(_VOXEL_SIZE[0] * _OUT_STRIDE)
    sy = float(_VOXEL_SIZE[1] * _OUT_STRIDE)
    ox = float(_PC_START[0])
    oy = float(_PC_START[1])

    mesh = plsc.VectorSubcoreMesh(core_axis_name="c", subcore_axis_name="s")

    @functools.partial(
        pl.kernel,
        out_type=jax.ShapeDtypeStruct((B * TN * TPC * 8, _RL), jnp.float32),
        mesh=mesh,
        compiler_params=pltpu.CompilerParams(use_tc_tiling_on_sc=False,
                                             needs_layout_passes=False),
        scratch_types=[
            pltpu.VMEM((P, 2, npw), jnp.float32),
            pltpu.VMEM((ppw,), jnp.float32),
            pltpu.VMEM((ppw,), jnp.float32),
            pltpu.VMEM((ppw,), jnp.float32),
            pltpu.VMEM((ppw,), jnp.float32),
            pltpu.VMEM((nchunk // 2, 16 * _LANES), jnp.int32),
            pltpu.VMEM((nchunk // 2, 4 * _LANES), jnp.int32),
            pltpu.VMEM((16 * _LANES, _RL), jnp.float32),
            pltpu.VMEM((16 * _LANES, _RL), jnp.float32),
            pltpu.VMEM((4 * _LANES, _RL), jnp.float32),
            pltpu.VMEM((4 * _LANES, _RL), jnp.float32),
            pltpu.SemaphoreType.DMA,
            pltpu.SemaphoreType.DMA,
            pltpu.SemaphoreType.DMA,
            pltpu.SemaphoreType.DMA,
        ],
    )
    def bev_kernel(rows_hbm, bc_hbm, out_hbm,
                   bcb, wab, wbb, wcb, wdb, gidx, oidx,
                   gb0, gb1, ob0, ob1, gs0, gs1, os0, os1):
        wid = lax.axis_index("s") * NC + lax.axis_index("c")
        b = wid // wpb
        nb = wid % wpb
        n0 = nb * npw

        pltpu.sync_copy(bc_hbm.at[b, :, nb], bcb)

        lane = lax.iota(jnp.int32, _LANES)

        nc_per_p = npw // _LANES

        def stage_a(ci, _):
            p = ci // nc_per_p
            nc = ci - p * nc_per_p
            sn = pl.ds(nc * _LANES, _LANES)
            x = (bcb[p, 0, sn] - ox) / sx
            y = (bcb[p, 1, sn] - oy) / sy
            xi = x.astype(jnp.int32)
            yi = y.astype(jnp.int32)
            xi = jnp.where(xi.astype(jnp.float32) > x, xi - 1, xi)
            yi = jnp.where(yi.astype(jnp.float32) > y, yi - 1, yi)
            x0 = jnp.clip(xi, 0, W - 1)
            y0 = jnp.clip(yi, 0, H - 1)
            x1 = jnp.minimum(x0 + 1, W - 1)
            y1 = jnp.minimum(y0 + 1, H - 1)
            wx0 = x - x0.astype(jnp.float32)
            wx1 = x1.astype(jnp.float32) - x
            wy0 = y - y0.astype(jnp.float32)
            wy1 = y1.astype(jnp.float32) - y
            s = pl.ds(ci * _LANES, _LANES)
            wab[s] = wx1 * wy1
            wbb[s] = wx1 * wy0
            wcb[s] = wx0 * wy1
            wdb[s] = wx0 * wy0
            ra = (y0 * W + x0) * (CT * B) + b
            rb = (y1 * W + x0) * (CT * B) + b
            rc = (y0 * W + x1) * (CT * B) + b
            rd = (y1 * W + x1) * (CT * B) + b
            c2 = ci // 2
            g0 = (ci - c2 * 2) * (8 * _LANES)
            gidx[c2, pl.ds(g0 + 0 * _LANES, _LANES)] = ra
            gidx[c2, pl.ds(g0 + 1 * _LANES, _LANES)] = ra + B
            gidx[c2, pl.ds(g0 + 2 * _LANES, _LANES)] = rb
            gidx[c2, pl.ds(g0 + 3 * _LANES, _LANES)] = rb + B
            gidx[c2, pl.ds(g0 + 4 * _LANES, _LANES)] = rc
            gidx[c2, pl.ds(g0 + 5 * _LANES, _LANES)] = rc + B
            gidx[c2, pl.ds(g0 + 6 * _LANES, _LANES)] = rd
            gidx[c2, pl.ds(g0 + 7 * _LANES, _LANES)] = rd + B
            n = n0 + nc * _LANES + lane
            o0 = ((b * TN + lax.shift_right_logical(n, 3)) * TPC
                  + p * CT) * 8 + (n & 7)
            q0 = (ci - c2 * 2) * (2 * _LANES)
            oidx[c2, pl.ds(q0, _LANES)] = o0
            oidx[c2, pl.ds(q0 + _LANES, _LANES)] = o0 + 8

        def combine(c2, gbuf, obuf):
            def sub_body(sub, _):
                s = pl.ds((c2 * 2 + sub) * _LANES, _LANES)
                wa = wab[s]
                wb = wbb[s]
                wc = wcb[s]
                wd = wdb[s]
                gbase = sub * (8 * _LANES)
                obase = sub * (2 * _LANES)

                @plsc.parallel_loop(0, _LANES, step=1, unroll=1)
                def point(i):
                    bi = jnp.full((_LANES,), i, jnp.int32)
                    wai = wa.at[bi].get(mode="promise_in_bounds")
                    wbi = wb.at[bi].get(mode="promise_in_bounds")
                    wci = wc.at[bi].get(mode="promise_in_bounds")
                    wdi = wd.at[bi].get(mode="promise_in_bounds")
                    for t in range(CT):
                        r = t * _LANES + i
                        for v in range(_RL // _LANES):
                            cs = pl.ds(v * _LANES, _LANES)
                            acc = (gbuf[gbase + 0 * CT * _LANES + r, cs] * wai
                                   + gbuf[gbase + 1 * CT * _LANES + r, cs] * wbi
                                   + gbuf[gbase + 2 * CT * _LANES + r, cs] * wci
                                   + gbuf[gbase + 3 * CT * _LANES + r, cs] * wdi)
                            obuf[obase + r, cs] = acc
                return 0

            lax.fori_loop(0, 2, sub_body, 0)

        nbig = nchunk // 2
        last = nbig - 1
        lax.fori_loop(0, 4, stage_a, None)
        pltpu.async_copy(rows_hbm.at[gidx.at[0]], gb0, gs0)
        pltpu.async_copy(rows_hbm.at[gidx.at[1]], gb1, gs1)

        def pair(hi, _):
            for par, gbuf, gsem, obuf, osem in (
                    (0, gb0, gs0, ob0, os0), (1, gb1, gs1, ob1, os1)):
                c2 = hi * 2 + par
                pltpu.make_async_copy(rows_hbm.at[gidx.at[c2]], gbuf,
                                      gsem).wait()

                @pl.when(hi > 0)
                def _wait_prev_scatter(obuf=obuf, osem=osem, c2=c2):
                    pltpu.make_async_copy(obuf, out_hbm.at[oidx.at[c2 - 2]],
                                          osem).wait()

                combine(c2, gbuf, obuf)
                pltpu.async_copy(obuf, out_hbm.at[oidx.at[c2]], osem)

                @pl.when(c2 + 2 <= last)
                def _prep_next(c2=c2):
                    lax.fori_loop((c2 + 2) * 2, (c2 + 2) * 2 + 2, stage_a,
                                  None)

                nxt = jnp.minimum(c2 + 2, last)
                pltpu.async_copy(rows_hbm.at[gidx.at[nxt]], gbuf, gsem)
            return 0

        lax.fori_loop(0, nbig // 2, pair, 0)

        pltpu.make_async_copy(rows_hbm.at[gidx.at[last]], gb0, gs0).wait()
        pltpu.make_async_copy(rows_hbm.at[gidx.at[last]], gb1, gs1).wait()
        pltpu.make_async_copy(ob0, out_hbm.at[oidx.at[last - 1]], os0).wait()
        pltpu.make_async_copy(ob1, out_hbm.at[oidx.at[last]], os1).wait()

    return bev_kernel


def kernel(bev_feature, batch_centers, num_point):
    if isinstance(num_point, tuple):
        num_point = num_point[0] * num_point[1]
    B, C, H, W = bev_feature.shape
    _, N, P, _ = batch_centers.shape
    CT = C // _RL
    TN = N // 8
    TPC = P * C // _RL
    bev_rows = jnp.transpose(
        bev_feature.reshape(B, CT, _RL, H, W), (3, 4, 1, 0, 2)
    ).reshape(H * W * CT * B, _RL)
    bc_view = jnp.transpose(
        batch_centers.reshape(B, N // _RL, _RL, P, 2), (0, 3, 1, 4, 2))
    out_rows = _make_sc_kernel(B, C, H, W, N, P)(bev_rows, bc_view)
    return jnp.transpose(
        out_rows.reshape(B, TN, TPC, 8, _RL), (0, 1, 3, 2, 4)
    ).reshape(B, N, P * C)


# speedup vs baseline: 1.2303x; 1.0031x over previous
import functools

import jax
import jax.numpy as jnp
from jax import lax
from jax.experimental import pallas as pl
from jax.experimental.pallas import tpu as pltpu
from jax.experimental.pallas import tpu_sc as plsc

_PC_START = (-54.0, -54.0)
_VOXEL_SIZE = (0.075, 0.075)
_OUT_STRIDE = 8

_LANES = 16
_RL = 128


@functools.cache
def _make_sc_kernel(B, C, H, W, N, P):
    info = plsc.get_sparse_core_info()
    NC, NS = info.num_cores, info.num_subcores
    NW = NC * NS
    NPTS = N * P
    CT = C // _RL
    TN = N // 8
    TPC = P * C // _RL
    ppw = B * NPTS // NW
    nchunk = ppw // _LANES
    wpb = NW // B
    npw = N // wpb
    assert CT * _RL == C and TN * 8 == N and wpb * B == NW
    assert ppw * NW == B * NPTS and nchunk % 2 == 0
    assert npw == _RL and nchunk == P * (npw // _LANES)

    sx = float(_VOXEL_SIZE[0] * _OUT_STRIDE)
    sy = float(_VOXEL_SIZE[1] * _OUT_STRIDE)
    ox = float(_PC_START[0])
    oy = float(_PC_START[1])

    mesh = plsc.VectorSubcoreMesh(core_axis_name="c", subcore_axis_name="s")

    @functools.partial(
        pl.kernel,
        out_type=jax.ShapeDtypeStruct((B * TN * TPC * 8, _RL), jnp.float32),
        mesh=mesh,
        compiler_params=pltpu.CompilerParams(use_tc_tiling_on_sc=False,
                                             needs_layout_passes=False,
                                             disable_bounds_checks=True,
                                             skip_device_barrier=True),
        scratch_types=[
            pltpu.VMEM((P, 2, npw), jnp.float32),
            pltpu.VMEM((ppw,), jnp.float32),
            pltpu.VMEM((ppw,), jnp.float32),
            pltpu.VMEM((ppw,), jnp.float32),
            pltpu.VMEM((ppw,), jnp.float32),
            pltpu.VMEM((nchunk // 2, 16 * _LANES), jnp.int32),
            pltpu.VMEM((nchunk // 2, 4 * _LANES), jnp.int32),
            pltpu.VMEM((16 * _LANES, _RL), jnp.float32),
            pltpu.VMEM((16 * _LANES, _RL), jnp.float32),
            pltpu.VMEM((4 * _LANES, _RL), jnp.float32),
            pltpu.VMEM((4 * _LANES, _RL), jnp.float32),
            pltpu.SemaphoreType.DMA,
            pltpu.SemaphoreType.DMA,
            pltpu.SemaphoreType.DMA,
            pltpu.SemaphoreType.DMA,
        ],
    )
    def bev_kernel(rows_hbm, bc_hbm, out_hbm,
                   bcb, wab, wbb, wcb, wdb, gidx, oidx,
                   gb0, gb1, ob0, ob1, gs0, gs1, os0, os1):
        wid = lax.axis_index("s") * NC + lax.axis_index("c")
        b = wid // wpb
        nb = wid % wpb
        n0 = nb * npw

        pltpu.sync_copy(bc_hbm.at[b, :, nb], bcb)

        lane = lax.iota(jnp.int32, _LANES)

        nc_per_p = npw // _LANES

        def stage_a(ci, _):
            p = ci // nc_per_p
            nc = ci - p * nc_per_p
            sn = pl.ds(nc * _LANES, _LANES)
            x = (bcb[p, 0, sn] - ox) / sx
            y = (bcb[p, 1, sn] - oy) / sy
            xi = x.astype(jnp.int32)
            yi = y.astype(jnp.int32)
            xi = jnp.where(xi.astype(jnp.float32) > x, xi - 1, xi)
            yi = jnp.where(yi.astype(jnp.float32) > y, yi - 1, yi)
            x0 = jnp.clip(xi, 0, W - 1)
            y0 = jnp.clip(yi, 0, H - 1)
            x1 = jnp.minimum(x0 + 1, W - 1)
            y1 = jnp.minimum(y0 + 1, H - 1)
            wx0 = x - x0.astype(jnp.float32)
            wx1 = x1.astype(jnp.float32) - x
            wy0 = y - y0.astype(jnp.float32)
            wy1 = y1.astype(jnp.float32) - y
            s = pl.ds(ci * _LANES, _LANES)
            wab[s] = wx1 * wy1
            wbb[s] = wx1 * wy0
            wcb[s] = wx0 * wy1
            wdb[s] = wx0 * wy0
            ra = (y0 * W + x0) * (CT * B) + b
            rb = (y1 * W + x0) * (CT * B) + b
            rc = (y0 * W + x1) * (CT * B) + b
            rd = (y1 * W + x1) * (CT * B) + b
            c2 = ci // 2
            g0 = (ci - c2 * 2) * (8 * _LANES)
            gidx[c2, pl.ds(g0 + 0 * _LANES, _LANES)] = ra
            gidx[c2, pl.ds(g0 + 1 * _LANES, _LANES)] = ra + B
            gidx[c2, pl.ds(g0 + 2 * _LANES, _LANES)] = rb
            gidx[c2, pl.ds(g0 + 3 * _LANES, _LANES)] = rb + B
            gidx[c2, pl.ds(g0 + 4 * _LANES, _LANES)] = rc
            gidx[c2, pl.ds(g0 + 5 * _LANES, _LANES)] = rc + B
            gidx[c2, pl.ds(g0 + 6 * _LANES, _LANES)] = rd
            gidx[c2, pl.ds(g0 + 7 * _LANES, _LANES)] = rd + B
            n = n0 + nc * _LANES + lane
            o0 = ((b * TN + lax.shift_right_logical(n, 3)) * TPC
                  + p * CT) * 8 + (n & 7)
            q0 = (ci - c2 * 2) * (2 * _LANES)
            oidx[c2, pl.ds(q0, _LANES)] = o0
            oidx[c2, pl.ds(q0 + _LANES, _LANES)] = o0 + 8

        def combine(c2, gbuf, obuf):
            def sub_body(sub, _):
                s = pl.ds((c2 * 2 + sub) * _LANES, _LANES)
                wa = wab[s]
                wb = wbb[s]
                wc = wcb[s]
                wd = wdb[s]
                gbase = sub * (8 * _LANES)
                obase = sub * (2 * _LANES)

                @plsc.parallel_loop(0, _LANES, step=1, unroll=1)
                def point(i):
                    bi = jnp.full((_LANES,), i, jnp.int32)
                    wai = wa.at[bi].get(mode="promise_in_bounds")
                    wbi = wb.at[bi].get(mode="promise_in_bounds")
                    wci = wc.at[bi].get(mode="promise_in_bounds")
                    wdi = wd.at[bi].get(mode="promise_in_bounds")
                    for t in range(CT):
                        r = t * _LANES + i
                        for v in range(_RL // _LANES):
                            cs = pl.ds(v * _LANES, _LANES)
                            acc = (gbuf[gbase + 0 * CT * _LANES + r, cs] * wai
                                   + gbuf[gbase + 1 * CT * _LANES + r, cs] * wbi
                                   + gbuf[gbase + 2 * CT * _LANES + r, cs] * wci
                                   + gbuf[gbase + 3 * CT * _LANES + r, cs] * wdi)
                            obuf[obase + r, cs] = acc
                return 0

            lax.fori_loop(0, 2, sub_body, 0)

        nbig = nchunk // 2
        last = nbig - 1
        lax.fori_loop(0, 4, stage_a, None)
        pltpu.async_copy(rows_hbm.at[gidx.at[0]], gb0, gs0)
        pltpu.async_copy(rows_hbm.at[gidx.at[1]], gb1, gs1)

        def pair(hi, _):
            for par, gbuf, gsem, obuf, osem in (
                    (0, gb0, gs0, ob0, os0), (1, gb1, gs1, ob1, os1)):
                c2 = hi * 2 + par
                pltpu.make_async_copy(rows_hbm.at[gidx.at[c2]], gbuf,
                                      gsem).wait()

                @pl.when(hi > 0)
                def _wait_prev_scatter(obuf=obuf, osem=osem, c2=c2):
                    pltpu.make_async_copy(obuf, out_hbm.at[oidx.at[c2 - 2]],
                                          osem).wait()

                combine(c2, gbuf, obuf)
                pltpu.async_copy(obuf, out_hbm.at[oidx.at[c2]], osem)

                @pl.when(c2 + 2 <= last)
                def _prep_next(c2=c2):
                    lax.fori_loop((c2 + 2) * 2, (c2 + 2) * 2 + 2, stage_a,
                                  None)

                nxt = jnp.minimum(c2 + 2, last)
                pltpu.async_copy(rows_hbm.at[gidx.at[nxt]], gbuf, gsem)
            return 0

        lax.fori_loop(0, nbig // 2, pair, 0)

        pltpu.make_async_copy(rows_hbm.at[gidx.at[last]], gb0, gs0).wait()
        pltpu.make_async_copy(rows_hbm.at[gidx.at[last]], gb1, gs1).wait()
        pltpu.make_async_copy(ob0, out_hbm.at[oidx.at[last - 1]], os0).wait()
        pltpu.make_async_copy(ob1, out_hbm.at[oidx.at[last]], os1).wait()

    return bev_kernel


def kernel(bev_feature, batch_centers, num_point):
    if isinstance(num_point, tuple):
        num_point = num_point[0] * num_point[1]
    B, C, H, W = bev_feature.shape
    _, N, P, _ = batch_centers.shape
    CT = C // _RL
    TN = N // 8
    TPC = P * C // _RL
    bev_rows = jnp.transpose(
        bev_feature.reshape(B, CT, _RL, H, W), (3, 4, 1, 0, 2)
    ).reshape(H * W * CT * B, _RL)
    bc_view = jnp.transpose(
        batch_centers.reshape(B, N // _RL, _RL, P, 2), (0, 3, 1, 4, 2))
    out_rows = _make_sc_kernel(B, C, H, W, N, P)(bev_rows, bc_view)
    return jnp.transpose(
        out_rows.reshape(B, TN, TPC, 8, _RL), (0, 1, 3, 2, 4)
    ).reshape(B, N, P * C)


# speedup vs baseline: 1.2338x; 1.0029x over previous
import functools

import jax
import jax.numpy as jnp
from jax import lax
from jax.experimental import pallas as pl
from jax.experimental.pallas import tpu as pltpu
from jax.experimental.pallas import tpu_sc as plsc

_PC_START = (-54.0, -54.0)
_VOXEL_SIZE = (0.075, 0.075)
_OUT_STRIDE = 8

_LANES = 16
_RL = 128


@functools.cache
def _make_sc_kernel(B, C, H, W, N, P):
    info = plsc.get_sparse_core_info()
    NC, NS = info.num_cores, info.num_subcores
    NW = NC * NS
    NPTS = N * P
    CT = C // _RL
    TN = N // 8
    TPC = P * C // _RL
    ppw = B * NPTS // NW
    nchunk = ppw // _LANES
    wpb = NW // B
    npw = N // wpb
    assert CT * _RL == C and TN * 8 == N and wpb * B == NW
    assert ppw * NW == B * NPTS and nchunk % 2 == 0
    assert npw == _RL and nchunk == P * (npw // _LANES)

    sx = float(_VOXEL_SIZE[0] * _OUT_STRIDE)
    sy = float(_VOXEL_SIZE[1] * _OUT_STRIDE)
    ox = float(_PC_START[0])
    oy = float(_PC_START[1])

    mesh = plsc.VectorSubcoreMesh(core_axis_name="c", subcore_axis_name="s")

    @functools.partial(
        pl.kernel,
        out_type=jax.ShapeDtypeStruct((B * TN * TPC * 8, _RL), jnp.float32),
        mesh=mesh,
        compiler_params=pltpu.CompilerParams(use_tc_tiling_on_sc=False,
                                             needs_layout_passes=False,
                                             disable_bounds_checks=True,
                                             skip_device_barrier=True),
        scratch_types=[
            pltpu.VMEM((P, 2, npw), jnp.float32),
            pltpu.VMEM((ppw,), jnp.float32),
            pltpu.VMEM((ppw,), jnp.float32),
            pltpu.VMEM((ppw,), jnp.float32),
            pltpu.VMEM((ppw,), jnp.float32),
            pltpu.VMEM((nchunk // 2, 16 * _LANES), jnp.int32),
            pltpu.VMEM((nchunk // 2, 4 * _LANES), jnp.int32),
            pltpu.VMEM((16 * _LANES, _RL), jnp.float32),
            pltpu.VMEM((16 * _LANES, _RL), jnp.float32),
            pltpu.VMEM((4 * _LANES, _RL), jnp.float32),
            pltpu.VMEM((4 * _LANES, _RL), jnp.float32),
            pltpu.SemaphoreType.DMA,
            pltpu.SemaphoreType.DMA,
            pltpu.SemaphoreType.DMA,
            pltpu.SemaphoreType.DMA,
        ],
    )
    def bev_kernel(rows_hbm, bc_hbm, out_hbm,
                   bcb, wab, wbb, wcb, wdb, gidx, oidx,
                   gb0, gb1, ob0, ob1, gs0, gs1, os0, os1):
        wid = lax.axis_index("s") * NC + lax.axis_index("c")
        b = wid // wpb
        nb = wid % wpb
        n0 = nb * npw

        pltpu.sync_copy(bc_hbm.at[b, :, nb], bcb)

        lane = lax.iota(jnp.int32, _LANES)

        nc_per_p = npw // _LANES

        def stage_a(ci, _):
            p = ci // nc_per_p
            nc = ci - p * nc_per_p
            sn = pl.ds(nc * _LANES, _LANES)
            x = (bcb[p, 0, sn] - ox) / sx
            y = (bcb[p, 1, sn] - oy) / sy
            xi = x.astype(jnp.int32)
            yi = y.astype(jnp.int32)
            xi = jnp.where(xi.astype(jnp.float32) > x, xi - 1, xi)
            yi = jnp.where(yi.astype(jnp.float32) > y, yi - 1, yi)
            x0 = jnp.clip(xi, 0, W - 1)
            y0 = jnp.clip(yi, 0, H - 1)
            x1 = jnp.minimum(x0 + 1, W - 1)
            y1 = jnp.minimum(y0 + 1, H - 1)
            wx0 = x - x0.astype(jnp.float32)
            wx1 = x1.astype(jnp.float32) - x
            wy0 = y - y0.astype(jnp.float32)
            wy1 = y1.astype(jnp.float32) - y
            s = pl.ds(ci * _LANES, _LANES)
            wab[s] = wx1 * wy1
            wbb[s] = wx1 * wy0
            wcb[s] = wx0 * wy1
            wdb[s] = wx0 * wy0
            ra = (y0 * W + x0) * (CT * B) + b
            rb = (y1 * W + x0) * (CT * B) + b
            rc = (y0 * W + x1) * (CT * B) + b
            rd = (y1 * W + x1) * (CT * B) + b
            c2 = ci // 2
            g0 = (ci - c2 * 2) * (8 * _LANES)
            gidx[c2, pl.ds(g0 + 0 * _LANES, _LANES)] = ra
            gidx[c2, pl.ds(g0 + 1 * _LANES, _LANES)] = ra + B
            gidx[c2, pl.ds(g0 + 2 * _LANES, _LANES)] = rb
            gidx[c2, pl.ds(g0 + 3 * _LANES, _LANES)] = rb + B
            gidx[c2, pl.ds(g0 + 4 * _LANES, _LANES)] = rc
            gidx[c2, pl.ds(g0 + 5 * _LANES, _LANES)] = rc + B
            gidx[c2, pl.ds(g0 + 6 * _LANES, _LANES)] = rd
            gidx[c2, pl.ds(g0 + 7 * _LANES, _LANES)] = rd + B
            n = n0 + nc * _LANES + lane
            o0 = ((b * TN + lax.shift_right_logical(n, 3)) * TPC
                  + p * CT) * 8 + (n & 7)
            q0 = (ci - c2 * 2) * (2 * _LANES)
            oidx[c2, pl.ds(q0, _LANES)] = o0
            oidx[c2, pl.ds(q0 + _LANES, _LANES)] = o0 + 8

        def combine(c2, gbuf, obuf):
            def sub_body(sub, _):
                s = pl.ds((c2 * 2 + sub) * _LANES, _LANES)
                wa = wab[s]
                wb = wbb[s]
                wc = wcb[s]
                wd = wdb[s]
                gbase = sub * (8 * _LANES)
                obase = sub * (2 * _LANES)

                @plsc.parallel_loop(0, _LANES, step=1, unroll=1)
                def point(i):
                    bi = jnp.full((_LANES,), i, jnp.int32)
                    wai = wa.at[bi].get(mode="promise_in_bounds")
                    wbi = wb.at[bi].get(mode="promise_in_bounds")
                    wci = wc.at[bi].get(mode="promise_in_bounds")
                    wdi = wd.at[bi].get(mode="promise_in_bounds")
                    for t in range(CT):
                        r = t * _LANES + i
                        for v in range(_RL // _LANES):
                            cs = pl.ds(v * _LANES, _LANES)
                            acc = (gbuf[gbase + 0 * CT * _LANES + r, cs] * wai
                                   + gbuf[gbase + 1 * CT * _LANES + r, cs] * wbi
                                   + gbuf[gbase + 2 * CT * _LANES + r, cs] * wci
                                   + gbuf[gbase + 3 * CT * _LANES + r, cs] * wdi)
                            obuf[obase + r, cs] = acc
                return 0

            lax.fori_loop(0, 2, sub_body, 0)

        nbig = nchunk // 2
        last = nbig - 1
        lax.fori_loop(0, 6, stage_a, None)
        pltpu.async_copy(rows_hbm.at[gidx.at[0]], gb0, gs0)
        pltpu.async_copy(rows_hbm.at[gidx.at[1]], gb1, gs1)

        def pair(hi, _):
            for par, gbuf, gsem, obuf, osem in (
                    (0, gb0, gs0, ob0, os0), (1, gb1, gs1, ob1, os1)):
                c2 = hi * 2 + par
                pltpu.make_async_copy(rows_hbm.at[gidx.at[c2]], gbuf,
                                      gsem).wait()

                @pl.when(hi > 0)
                def _wait_prev_scatter(obuf=obuf, osem=osem, c2=c2):
                    pltpu.make_async_copy(obuf, out_hbm.at[oidx.at[c2 - 2]],
                                          osem).wait()

                combine(c2, gbuf, obuf)
                pltpu.async_copy(obuf, out_hbm.at[oidx.at[c2]], osem)
                nxt = jnp.minimum(c2 + 2, last)
                pltpu.async_copy(rows_hbm.at[gidx.at[nxt]], gbuf, gsem)

                @pl.when(c2 + 3 <= last)
                def _prep_next(c2=c2):
                    lax.fori_loop((c2 + 3) * 2, (c2 + 3) * 2 + 2, stage_a,
                                  None)
            return 0

        lax.fori_loop(0, nbig // 2, pair, 0)

        pltpu.make_async_copy(rows_hbm.at[gidx.at[last]], gb0, gs0).wait()
        pltpu.make_async_copy(rows_hbm.at[gidx.at[last]], gb1, gs1).wait()
        pltpu.make_async_copy(ob0, out_hbm.at[oidx.at[last - 1]], os0).wait()
        pltpu.make_async_copy(ob1, out_hbm.at[oidx.at[last]], os1).wait()

    return bev_kernel


def kernel(bev_feature, batch_centers, num_point):
    if isinstance(num_point, tuple):
        num_point = num_point[0] * num_point[1]
    B, C, H, W = bev_feature.shape
    _, N, P, _ = batch_centers.shape
    CT = C // _RL
    TN = N // 8
    TPC = P * C // _RL
    bev_rows = jnp.transpose(
        bev_feature.reshape(B, CT, _RL, H, W), (3, 4, 1, 0, 2)
    ).reshape(H * W * CT * B, _RL)
    bc_view = jnp.transpose(
        batch_centers.reshape(B, N // _RL, _RL, P, 2), (0, 3, 1, 4, 2))
    out_rows = _make_sc_kernel(B, C, H, W, N, P)(bev_rows, bc_view)
    return jnp.transpose(
        out_rows.reshape(B, TN, TPC, 8, _RL), (0, 1, 3, 2, 4)
    ).reshape(B, N, P * C)


# speedup vs baseline: 1.2365x; 1.0021x over previous
import functools

import jax
import jax.numpy as jnp
from jax import lax
from jax.experimental import pallas as pl
from jax.experimental.pallas import tpu as pltpu
from jax.experimental.pallas import tpu_sc as plsc

_PC_START = (-54.0, -54.0)
_VOXEL_SIZE = (0.075, 0.075)
_OUT_STRIDE = 8

_LANES = 16
_RL = 128


@functools.cache
def _make_sc_kernel(B, C, H, W, N, P):
    info = plsc.get_sparse_core_info()
    NC, NS = info.num_cores, info.num_subcores
    NW = NC * NS
    NPTS = N * P
    CT = C // _RL
    TN = N // 8
    TPC = P * C // _RL
    ppw = B * NPTS // NW
    nchunk = ppw // _LANES
    wpb = NW // B
    npw = N // wpb
    assert CT * _RL == C and TN * 8 == N and wpb * B == NW
    assert ppw * NW == B * NPTS and nchunk % 2 == 0
    assert npw == _RL and nchunk == P * (npw // _LANES)

    sx = float(_VOXEL_SIZE[0] * _OUT_STRIDE)
    sy = float(_VOXEL_SIZE[1] * _OUT_STRIDE)
    ox = float(_PC_START[0])
    oy = float(_PC_START[1])

    mesh = plsc.VectorSubcoreMesh(core_axis_name="c", subcore_axis_name="s")

    @functools.partial(
        pl.kernel,
        out_type=jax.ShapeDtypeStruct((B * TN * TPC * 8, _RL), jnp.float32),
        mesh=mesh,
        compiler_params=pltpu.CompilerParams(use_tc_tiling_on_sc=False,
                                             needs_layout_passes=False,
                                             disable_bounds_checks=True,
                                             skip_device_barrier=True),
        scratch_types=[
            pltpu.VMEM((P, 2, npw), jnp.float32),
            pltpu.VMEM((ppw,), jnp.float32),
            pltpu.VMEM((ppw,), jnp.float32),
            pltpu.VMEM((ppw,), jnp.float32),
            pltpu.VMEM((ppw,), jnp.float32),
            pltpu.VMEM((nchunk // 2, 16 * _LANES), jnp.int32),
            pltpu.VMEM((nchunk // 2, 4 * _LANES), jnp.int32),
            pltpu.VMEM((16 * _LANES, _RL), jnp.float32),
            pltpu.VMEM((16 * _LANES, _RL), jnp.float32),
            pltpu.VMEM((4 * _LANES, _RL), jnp.float32),
            pltpu.VMEM((4 * _LANES, _RL), jnp.float32),
            pltpu.SemaphoreType.DMA,
            pltpu.SemaphoreType.DMA,
            pltpu.SemaphoreType.DMA,
            pltpu.SemaphoreType.DMA,
        ],
    )
    def bev_kernel(rows_hbm, bc_hbm, out_hbm,
                   bcb, wab, wbb, wcb, wdb, gidx, oidx,
                   gb0, gb1, ob0, ob1, gs0, gs1, os0, os1):
        wid = lax.axis_index("s") * NC + lax.axis_index("c")
        b = wid // wpb
        nb = wid % wpb
        n0 = nb * npw

        pltpu.sync_copy(bc_hbm.at[b, :, nb], bcb)

        lane = lax.iota(jnp.int32, _LANES)

        nc_per_p = npw // _LANES

        def stage_a(ci, _):
            p = ci // nc_per_p
            nc = ci - p * nc_per_p
            sn = pl.ds(nc * _LANES, _LANES)
            x = (bcb[p, 0, sn] - ox) / sx
            y = (bcb[p, 1, sn] - oy) / sy
            xi = x.astype(jnp.int32)
            yi = y.astype(jnp.int32)
            xi = jnp.where(xi.astype(jnp.float32) > x, xi - 1, xi)
            yi = jnp.where(yi.astype(jnp.float32) > y, yi - 1, yi)
            x0 = jnp.clip(xi, 0, W - 1)
            y0 = jnp.clip(yi, 0, H - 1)
            x1 = jnp.minimum(x0 + 1, W - 1)
            y1 = jnp.minimum(y0 + 1, H - 1)
            wx0 = x - x0.astype(jnp.float32)
            wx1 = x1.astype(jnp.float32) - x
            wy0 = y - y0.astype(jnp.float32)
            wy1 = y1.astype(jnp.float32) - y
            s = pl.ds(ci * _LANES, _LANES)
            wab[s] = wx1 * wy1
            wbb[s] = wx1 * wy0
            wcb[s] = wx0 * wy1
            wdb[s] = wx0 * wy0
            ra = (y0 * W + x0) * (CT * B) + b
            rb = (y1 * W + x0) * (CT * B) + b
            rc = (y0 * W + x1) * (CT * B) + b
            rd = (y1 * W + x1) * (CT * B) + b
            c2 = ci // 2
            g0 = (ci - c2 * 2) * (8 * _LANES)
            gidx[c2, pl.ds(g0 + 0 * _LANES, _LANES)] = ra
            gidx[c2, pl.ds(g0 + 1 * _LANES, _LANES)] = ra + B
            gidx[c2, pl.ds(g0 + 2 * _LANES, _LANES)] = rb
            gidx[c2, pl.ds(g0 + 3 * _LANES, _LANES)] = rb + B
            gidx[c2, pl.ds(g0 + 4 * _LANES, _LANES)] = rc
            gidx[c2, pl.ds(g0 + 5 * _LANES, _LANES)] = rc + B
            gidx[c2, pl.ds(g0 + 6 * _LANES, _LANES)] = rd
            gidx[c2, pl.ds(g0 + 7 * _LANES, _LANES)] = rd + B
            n = n0 + nc * _LANES + lane
            o0 = ((b * TN + lax.shift_right_logical(n, 3)) * TPC
                  + p * CT) * 8 + (n & 7)
            q0 = (ci - c2 * 2) * (2 * _LANES)
            oidx[c2, pl.ds(q0, _LANES)] = o0
            oidx[c2, pl.ds(q0 + _LANES, _LANES)] = o0 + 8

        def combine(c2, gbuf, obuf):
            def sub_body(sub, _):
                s = pl.ds((c2 * 2 + sub) * _LANES, _LANES)
                wa = wab[s]
                wb = wbb[s]
                wc = wcb[s]
                wd = wdb[s]
                gbase = sub * (8 * _LANES)
                obase = sub * (2 * _LANES)

                @plsc.parallel_loop(0, _LANES, step=1, unroll=1)
                def point(i):
                    bi = jnp.full((_LANES,), i, jnp.int32)
                    wai = wa.at[bi].get(mode="promise_in_bounds")
                    wbi = wb.at[bi].get(mode="promise_in_bounds")
                    wci = wc.at[bi].get(mode="promise_in_bounds")
                    wdi = wd.at[bi].get(mode="promise_in_bounds")
                    for t in range(CT):
                        r = t * _LANES + i
                        for v in range(_RL // _LANES):
                            cs = pl.ds(v * _LANES, _LANES)
                            acc = (gbuf[gbase + 0 * CT * _LANES + r, cs] * wai
                                   + gbuf[gbase + 1 * CT * _LANES + r, cs] * wbi
                                   + gbuf[gbase + 2 * CT * _LANES + r, cs] * wci
                                   + gbuf[gbase + 3 * CT * _LANES + r, cs] * wdi)
                            obuf[obase + r, cs] = acc
                return 0

            lax.fori_loop(0, 2, sub_body, 0)

        nbig = nchunk // 2
        last = nbig - 1
        lax.fori_loop(0, 2, stage_a, None)
        pltpu.async_copy(rows_hbm.at[gidx.at[0]], gb0, gs0)
        lax.fori_loop(2, 4, stage_a, None)
        pltpu.async_copy(rows_hbm.at[gidx.at[1]], gb1, gs1)
        lax.fori_loop(4, 6, stage_a, None)

        def pair(hi, _):
            for par, gbuf, gsem, obuf, osem in (
                    (0, gb0, gs0, ob0, os0), (1, gb1, gs1, ob1, os1)):
                c2 = hi * 2 + par
                pltpu.make_async_copy(rows_hbm.at[gidx.at[c2]], gbuf,
                                      gsem).wait()

                @pl.when(hi > 0)
                def _wait_prev_scatter(obuf=obuf, osem=osem, c2=c2):
                    pltpu.make_async_copy(obuf, out_hbm.at[oidx.at[c2 - 2]],
                                          osem).wait()

                combine(c2, gbuf, obuf)
                pltpu.async_copy(obuf, out_hbm.at[oidx.at[c2]], osem)
                nxt = jnp.minimum(c2 + 2, last)
                pltpu.async_copy(rows_hbm.at[gidx.at[nxt]], gbuf, gsem)

                @pl.when(c2 + 3 <= last)
                def _prep_next(c2=c2):
                    lax.fori_loop((c2 + 3) * 2, (c2 + 3) * 2 + 2, stage_a,
                                  None)
            return 0

        lax.fori_loop(0, nbig // 2, pair, 0)

        pltpu.make_async_copy(rows_hbm.at[gidx.at[last]], gb0, gs0).wait()
        pltpu.make_async_copy(rows_hbm.at[gidx.at[last]], gb1, gs1).wait()
        pltpu.make_async_copy(ob0, out_hbm.at[oidx.at[last - 1]], os0).wait()
        pltpu.make_async_copy(ob1, out_hbm.at[oidx.at[last]], os1).wait()

    return bev_kernel


def kernel(bev_feature, batch_centers, num_point):
    if isinstance(num_point, tuple):
        num_point = num_point[0] * num_point[1]
    B, C, H, W = bev_feature.shape
    _, N, P, _ = batch_centers.shape
    CT = C // _RL
    TN = N // 8
    TPC = P * C // _RL
    bev_rows = jnp.transpose(
        bev_feature.reshape(B, CT, _RL, H, W), (3, 4, 1, 0, 2)
    ).reshape(H * W * CT * B, _RL)
    bc_view = jnp.transpose(
        batch_centers.reshape(B, N // _RL, _RL, P, 2), (0, 3, 1, 4, 2))
    out_rows = _make_sc_kernel(B, C, H, W, N, P)(bev_rows, bc_view)
    return jnp.transpose(
        out_rows.reshape(B, TN, TPC, 8, _RL), (0, 1, 3, 2, 4)
    ).reshape(B, N, P * C)
